# Initial kernel scaffold; baseline (speedup 1.0000x reference)
import jax
import jax.numpy as jnp
from jax.experimental import pallas as pl


def kernel(scores):
    raise NotImplementedError("write your pallas kernel here")


# speedup vs baseline: 25.3937x; 25.3937x over previous
import functools
import math

import jax
import jax.numpy as jnp
from jax.experimental import pallas as pl
from jax.experimental.pallas import tpu as pltpu

_LARGE_NUMBER = 1e10
_NEG = -1e30
_K = 32
_S = 2
_ROWS_PER_BLOCK = 1024


def _simple_body(th_ref, u_ref, marg_ref, masks_ref, bscr, qscr, *, n, kp1):
    f32 = jnp.float32
    neg_row = jnp.full((1, 8, 128), _NEG, f32)
    binit = jnp.concatenate(
        [jnp.zeros((1, 8, 128), f32), jnp.full((kp1 - 1, 8, 128), _NEG, f32)], axis=0)
    bscr[n] = binit

    def bstep(t, bnext):
        i = n - 1 - t
        th_i = th_ref[pl.ds(i, 1), 0]
        shifted = jnp.concatenate([neg_row, bnext[:-1]], axis=0)
        lognum = th_i + shifted
        bi = jnp.logaddexp(bnext, lognum)
        bscr[pl.ds(i, 1)] = bi[None]
        qscr[pl.ds(i, 1)] = jnp.exp(lognum - bi)[None]
        return bi

    b0 = jax.lax.fori_loop(0, n, bstep, binit)
    log_z = b0[kp1 - 1]

    def mstep(i, f):
        th_i = th_ref[pl.ds(i, 1), 0]
        bias = th_i[0] - log_z
        bn = bscr[pl.ds(i + 1, 1)][0]
        acc = jnp.zeros((8, 128), f32)
        for j in range(kp1 - 1):
            acc = acc + jnp.exp(f[j] + bn[kp1 - 2 - j] + bias)
        marg_ref[pl.ds(i, 1)] = acc[None, None]
        shifted = jnp.concatenate([neg_row, f[:-1]], axis=0)
        return jnp.logaddexp(f, th_i + shifted)

    jax.lax.fori_loop(0, n, mstep, binit)

    jj = jax.lax.broadcasted_iota(jnp.int32, (kp1, 8, 128), 0)

    def sstep(i, rs):
        qi = qscr[pl.ds(i, 1)][0]
        out = []
        for s in range(_S):
            r = rs[s]
            p = jnp.sum(jnp.where(jj == r[None], qi, 0.0), axis=0)
            u = u_ref[pl.ds(_S * i + s, 1), 0][0]
            inc = u < p
            masks_ref[pl.ds(s * n + i, 1)] = inc.astype(f32)[None, None]
            out.append(r - inc.astype(jnp.int32))
        return tuple(out)

    r0 = jnp.full((8, 128), _K, jnp.int32)
    jax.lax.fori_loop(0, n, sstep, (r0,) * _S)


def kernel(scores):
    nnodes, choices, ensemble = scores.shape
    local_k = min(_K, choices)
    kp1 = local_k + 1
    n = 2 ** int(math.ceil(math.log2(choices)))
    rows = nnodes * ensemble
    rpb = _ROWS_PER_BLOCK
    nblocks = (rows + rpb - 1) // rpb
    rows_pad = nblocks * rpb

    flat = jnp.transpose(scores, (0, 2, 1)).reshape(rows, choices)
    if n > choices:
        flat = jnp.concatenate(
            [flat, jnp.full((rows, n - choices), -_LARGE_NUMBER, flat.dtype)], axis=1)
    th = jnp.pad(flat.T, ((0, 0), (0, rows_pad - rows)))
    th4 = th.reshape(n, nblocks, 8, 128)

    u = jax.random.uniform(jax.random.key(1), (n, _S, rows), dtype=scores.dtype)
    u4 = jnp.pad(u.reshape(n * _S, rows), ((0, 0), (0, rows_pad - rows)))
    u4 = u4.reshape(n * _S, nblocks, 8, 128)

    body = functools.partial(_simple_body, n=n, kp1=kp1)
    marg4, masks4 = pl.pallas_call(
        body,
        grid=(nblocks,),
        in_specs=[
            pl.BlockSpec((n, 1, 8, 128), lambda g: (0, g, 0, 0)),
            pl.BlockSpec((n * _S, 1, 8, 128), lambda g: (0, g, 0, 0)),
        ],
        out_specs=[
            pl.BlockSpec((n, 1, 8, 128), lambda g: (0, g, 0, 0)),
            pl.BlockSpec((_S * n, 1, 8, 128), lambda g: (0, g, 0, 0)),
        ],
        out_shape=[
            jax.ShapeDtypeStruct((n, nblocks, 8, 128), jnp.float32),
            jax.ShapeDtypeStruct((_S * n, nblocks, 8, 128), jnp.float32),
        ],
        scratch_shapes=[
            pltpu.VMEM((n + 1, kp1, 8, 128), jnp.float32),
            pltpu.VMEM((n, kp1, 8, 128), jnp.float32),
        ],
    )(th4, u4)

    marg = marg4.reshape(n, rows_pad)[:choices, :rows]
    marginals = marg.reshape(choices, nnodes, ensemble).transpose(1, 0, 2)

    masks = masks4.reshape(_S, n, rows_pad)[:, :choices, :rows]
    sb = masks.reshape(_S, choices, nnodes, ensemble).transpose(0, 2, 1, 3)
    samples = jax.lax.stop_gradient(sb - marginals[None]) + marginals[None]
    return samples, marginals


# speedup vs baseline: 33.5735x; 1.3221x over previous
import functools
import math

import jax
import jax.numpy as jnp
from jax import lax
from jax.experimental import pallas as pl
from jax.experimental.pallas import tpu as pltpu

_LARGE_NUMBER = 1e10
_NEG = -1e30
_K = 32
_S = 2
_ROWS_PER_BLOCK = 1024


def _laep(x1, x2):
    amax = lax.max(x1, x2)
    delta = lax.sub(x1, x2)
    return lax.add(amax, lax.log1p(lax.exp(lax.neg(lax.abs(delta)))))


def _simple_body(th_ref, u_ref, marg_ref, masks_ref, qscr, *, n, kp1):
    f32 = jnp.float32
    neg_row = jnp.full((1, 8, 128), _NEG, f32)
    binit = jnp.concatenate(
        [jnp.zeros((1, 8, 128), f32), jnp.full((kp1 - 1, 8, 128), _NEG, f32)], axis=0)

    def bstep(t, bnext):
        i = n - 1 - t
        th_i = th_ref[pl.ds(i, 1), 0]
        shifted = jnp.concatenate([neg_row, bnext[:-1]], axis=0)
        lognum = th_i + shifted
        bi = _laep(bnext, lognum)
        qscr[pl.ds(i, 1)] = jnp.exp(lognum - bi)[None]
        return bi

    jax.lax.fori_loop(0, n, bstep, binit)

    jj = lax.broadcasted_iota(jnp.int32, (kp1, 8, 128), 0)
    pi0 = jnp.concatenate(
        [jnp.zeros((kp1 - 1, 8, 128), f32), jnp.ones((1, 8, 128), f32)], axis=0)
    zero_row = jnp.zeros((1, 8, 128), f32)

    def fstep(i, carry):
        pi_v, *rs = carry
        qi = qscr[pl.ds(i, 1)][0]
        t = pi_v * qi
        marg_ref[pl.ds(i, 1)] = jnp.sum(t, axis=0)[None, None]
        pi_new = (pi_v - t) + jnp.concatenate([t[1:], zero_row], axis=0)
        out = [pi_new]
        for s in range(_S):
            r = rs[s]
            p = jnp.sum(jnp.where(jj == r[None], qi, 0.0), axis=0)
            u = u_ref[pl.ds(_S * i + s, 1), 0][0]
            inc = u < p
            masks_ref[pl.ds(s * n + i, 1)] = inc.astype(f32)[None, None]
            out.append(r - inc.astype(jnp.int32))
        return tuple(out)

    r0 = jnp.full((8, 128), _K, jnp.int32)
    jax.lax.fori_loop(0, n, fstep, (pi0,) + (r0,) * _S)


def kernel(scores):
    nnodes, choices, ensemble = scores.shape
    local_k = min(_K, choices)
    kp1 = local_k + 1
    n = 2 ** int(math.ceil(math.log2(choices)))
    rows = nnodes * ensemble
    rpb = _ROWS_PER_BLOCK
    nblocks = (rows + rpb - 1) // rpb
    rows_pad = nblocks * rpb

    th = jnp.transpose(scores, (1, 0, 2)).reshape(choices, rows)
    if n > choices:
        th = jnp.concatenate(
            [th, jnp.full((n - choices, rows), -_LARGE_NUMBER, th.dtype)], axis=0)
    th4 = jnp.pad(th, ((0, 0), (0, rows_pad - rows))).reshape(n, nblocks, 8, 128)

    u = jax.random.uniform(jax.random.key(1), (n, _S, rows), dtype=scores.dtype)
    u4 = jnp.pad(u.reshape(n * _S, rows), ((0, 0), (0, rows_pad - rows)))
    u4 = u4.reshape(n * _S, nblocks, 8, 128)

    body = functools.partial(_simple_body, n=n, kp1=kp1)
    marg4, masks4 = pl.pallas_call(
        body,
        grid=(nblocks,),
        in_specs=[
            pl.BlockSpec((n, 1, 8, 128), lambda g: (0, g, 0, 0)),
            pl.BlockSpec((n * _S, 1, 8, 128), lambda g: (0, g, 0, 0)),
        ],
        out_specs=[
            pl.BlockSpec((n, 1, 8, 128), lambda g: (0, g, 0, 0)),
            pl.BlockSpec((_S * n, 1, 8, 128), lambda g: (0, g, 0, 0)),
        ],
        out_shape=[
            jax.ShapeDtypeStruct((n, nblocks, 8, 128), jnp.float32),
            jax.ShapeDtypeStruct((_S * n, nblocks, 8, 128), jnp.float32),
        ],
        scratch_shapes=[
            pltpu.VMEM((n, kp1, 8, 128), jnp.float32),
        ],
    )(th4, u4)

    marg = marg4.reshape(n, rows_pad)[:choices, :rows]
    marginals = marg.reshape(choices, nnodes, ensemble).transpose(1, 0, 2)

    masks = masks4.reshape(_S, n, rows_pad)[:, :choices, :rows]
    sb = masks.reshape(_S, choices, nnodes, ensemble).transpose(0, 2, 1, 3)
    samples = jax.lax.stop_gradient(sb - marginals[None]) + marginals[None]
    return samples, marginals
